# baseline (speedup 1.0000x reference)
import jax
import jax.numpy as jnp
from jax import lax
from jax.experimental import pallas as pl
from jax.experimental.pallas import tpu as pltpu

_N = 4096
_D = 256
_EPS = 22.0
_MIN_SAMPLES = 5
_RB = 128
_G = _N // _RB
_CH = 512
_NCH = _N // _CH
_BIG = float(_N)


def _prop_c_to_r(adj_ref, c_ref, r_ref):
    c = c_ref[...]
    for k in range(_NCH):
        pen = adj_ref[:, k * _CH:(k + 1) * _CH].astype(jnp.float32)
        r_ref[0:1, k * _CH:(k + 1) * _CH] = jnp.min(
            c + pen * _BIG, axis=0, keepdims=True)


def _prop_r_to_c(adj_ref, r_ref, c_ref):
    r = r_ref[...]
    changed = jnp.zeros((), jnp.bool_)
    for k in range(_NCH):
        pen = adj_ref[k * _CH:(k + 1) * _CH, :].astype(jnp.float32)
        m = jnp.min(r + pen * _BIG, axis=1, keepdims=True)
        changed |= jnp.any(m != c_ref[k * _CH:(k + 1) * _CH, :])
        c_ref[k * _CH:(k + 1) * _CH, :] = m
    return changed


def _dbscan_kernel(xr_ref, x_ref, d_ref, lab_ref, adj_ref, sqc_ref, c0_ref,
                   r_ref):
    i = pl.program_id(0)

    @pl.when(i == 0)
    def _():
        xx = x_ref[...]
        sqc_ref[...] = lax.dot_general(
            jnp.ones((8, _D), jnp.float32), xx * xx,
            (((1,), (1,)), ((), ())),
            preferred_element_type=jnp.float32,
        )

    xr = xr_ref[...]
    xx = x_ref[...]
    dg = lax.dot_general(xr, xx, (((1,), (1,)), ((), ())),
                         preferred_element_type=jnp.float32,
                         precision=lax.Precision.HIGHEST)
    sqr = jnp.sum(xr * xr, axis=1, keepdims=True)
    sqc = sqc_ref[0:1, :]
    d2 = jnp.maximum(sqr + sqc - 2.0 * dg, 0.0)
    dblk = jnp.where(d2 > 0.0, jnp.sqrt(jnp.where(d2 > 0.0, d2, 1.0)), 0.0)
    d_ref[...] = dblk

    pen = (dblk > _EPS).astype(jnp.float32)
    adj_ref[pl.ds(i * _RB, _RB), :] = pen.astype(jnp.int8)

    iota = lax.broadcasted_iota(jnp.int32, (_RB, _N), 1).astype(jnp.float32)
    c0_ref[pl.ds(i * _RB, _RB), :] = jnp.min(
        iota + pen * _BIG, axis=1, keepdims=True)

    @pl.when(i == _G - 1)
    def _():
        def cond(changed):
            return changed

        def body(_):
            _prop_c_to_r(adj_ref, c0_ref, r_ref)
            return _prop_r_to_c(adj_ref, r_ref, c0_ref)

        lax.while_loop(cond, body, jnp.bool_(True))
        _prop_c_to_r(adj_ref, c0_ref, r_ref)
        cfin = c0_ref[...]

        s = jnp.zeros((_N, 1), jnp.float32)
        for k in range(_NCH):
            rr = r_ref[0:1, k * _CH:(k + 1) * _CH]
            eq = jnp.maximum(1.0 - jnp.abs(cfin - rr), 0.0)
            s = s + jnp.sum(eq, axis=1, keepdims=True)

        lab = jnp.where(s >= float(_MIN_SAMPLES), cfin, -1.0)
        lab_ref[...] = lab.astype(jnp.int32)


def kernel(X):
    d, labels = pl.pallas_call(
        _dbscan_kernel,
        grid=(_G,),
        in_specs=[
            pl.BlockSpec((_RB, _D), lambda i: (i, 0)),
            pl.BlockSpec((_N, _D), lambda i: (0, 0)),
        ],
        out_specs=[
            pl.BlockSpec((_RB, _N), lambda i: (i, 0)),
            pl.BlockSpec((_N, 1), lambda i: (0, 0)),
        ],
        out_shape=[
            jax.ShapeDtypeStruct((_N, _N), jnp.float32),
            jax.ShapeDtypeStruct((_N, 1), jnp.int32),
        ],
        scratch_shapes=[
            pltpu.VMEM((_N, _N), jnp.int8),
            pltpu.VMEM((8, _N), jnp.float32),
            pltpu.VMEM((_N, 1), jnp.float32),
            pltpu.VMEM((1, _N), jnp.float32),
        ],
        compiler_params=pltpu.CompilerParams(
            dimension_semantics=("arbitrary",),
        ),
    )(X, X)
    return d, labels.reshape(_N)


# speedup vs baseline: 1.7782x; 1.7782x over previous
import jax
import jax.numpy as jnp
from jax import lax
from jax.experimental import pallas as pl
from jax.experimental.pallas import tpu as pltpu

_N = 4096
_D = 256
_EPS = 22.0
_MIN_SAMPLES = 5
_RB = 128
_G = _N // _RB
_CH = 512
_NCH = _N // _CH
_BIG = float(_N)


def _prop_c_to_r(adj_ref, c_ref, r_ref):
    c = c_ref[...]
    for k in range(_NCH):
        pen = adj_ref[:, k * _CH:(k + 1) * _CH].astype(jnp.float32)
        r_ref[0:1, k * _CH:(k + 1) * _CH] = jnp.min(
            c + pen * _BIG, axis=0, keepdims=True)


def _prop_r_to_c(adj_ref, r_ref, c_ref):
    r = r_ref[...]
    changed = jnp.zeros((), jnp.bool_)
    for k in range(_NCH):
        pen = adj_ref[k * _CH:(k + 1) * _CH, :].astype(jnp.float32)
        m = jnp.min(r + pen * _BIG, axis=1, keepdims=True)
        changed |= jnp.any(m != c_ref[k * _CH:(k + 1) * _CH, :])
        c_ref[k * _CH:(k + 1) * _CH, :] = m
    return changed


def _dbscan_kernel(xr_ref, x_ref, d_ref, lab_ref, adj_ref, sqc_ref, c0_ref,
                   r_ref):
    i = pl.program_id(0)

    @pl.when(i == 0)
    def _():
        xx = x_ref[...]
        sqc_ref[...] = lax.dot_general(
            jnp.ones((8, _D), jnp.float32), xx * xx,
            (((1,), (1,)), ((), ())),
            preferred_element_type=jnp.float32,
        )

    xr = xr_ref[...]
    xx = x_ref[...]
    dg = lax.dot_general(xr, xx, (((1,), (1,)), ((), ())),
                         preferred_element_type=jnp.float32)
    sqr = jnp.sum(xr * xr, axis=1, keepdims=True)
    sqc = sqc_ref[0:1, :]
    d2 = jnp.maximum(sqr + sqc - 2.0 * dg, 0.0)
    dblk = jnp.where(d2 > 0.0, jnp.sqrt(jnp.where(d2 > 0.0, d2, 1.0)), 0.0)
    d_ref[...] = dblk

    pen = (dblk > _EPS).astype(jnp.float32)
    adj_ref[pl.ds(i * _RB, _RB), :] = pen.astype(jnp.int8)

    iota = lax.broadcasted_iota(jnp.int32, (_RB, _N), 1).astype(jnp.float32)
    c0_ref[pl.ds(i * _RB, _RB), :] = jnp.min(
        iota + pen * _BIG, axis=1, keepdims=True)

    @pl.when(i == _G - 1)
    def _():
        def cond(changed):
            return changed

        def body(_):
            _prop_c_to_r(adj_ref, c0_ref, r_ref)
            return _prop_r_to_c(adj_ref, r_ref, c0_ref)

        lax.while_loop(cond, body, jnp.bool_(True))
        cfin = c0_ref[...]

        s = jnp.zeros((_N, 1), jnp.float32)
        for k in range(_NCH):
            rr = r_ref[0:1, k * _CH:(k + 1) * _CH]
            eq = jnp.maximum(1.0 - jnp.abs(cfin - rr), 0.0)
            s = s + jnp.sum(eq, axis=1, keepdims=True)

        lab = jnp.where(s >= float(_MIN_SAMPLES), cfin, -1.0)
        lab_ref[...] = lab.astype(jnp.int32)


def kernel(X):
    d, labels = pl.pallas_call(
        _dbscan_kernel,
        grid=(_G,),
        in_specs=[
            pl.BlockSpec((_RB, _D), lambda i: (i, 0)),
            pl.BlockSpec((_N, _D), lambda i: (0, 0)),
        ],
        out_specs=[
            pl.BlockSpec((_RB, _N), lambda i: (i, 0)),
            pl.BlockSpec((_N, 1), lambda i: (0, 0)),
        ],
        out_shape=[
            jax.ShapeDtypeStruct((_N, _N), jnp.float32),
            jax.ShapeDtypeStruct((_N, 1), jnp.int32),
        ],
        scratch_shapes=[
            pltpu.VMEM((_N, _N), jnp.int8),
            pltpu.VMEM((8, _N), jnp.float32),
            pltpu.VMEM((_N, 1), jnp.float32),
            pltpu.VMEM((1, _N), jnp.float32),
        ],
        compiler_params=pltpu.CompilerParams(
            dimension_semantics=("arbitrary",),
        ),
    )(X, X)
    return d, labels.reshape(_N)


# speedup vs baseline: 2.3803x; 1.3386x over previous
import functools

import jax
import jax.numpy as jnp
from jax import lax
from jax.experimental import pallas as pl
from jax.experimental.pallas import tpu as pltpu
from jax.experimental.pallas import tpu_sc as plsc

_N = 4096
_D = 256
_EPS = 22.0
_MIN_SAMPLES = 5
_RB = 128
_G = _N // _RB
_CH = 512
_NCH = _N // _CH
_BIG = float(_N)


def _prop_c_to_r(adj_ref, c_ref, r_ref):
    c = c_ref[...]
    for k in range(_NCH):
        pen = adj_ref[:, k * _CH:(k + 1) * _CH].astype(jnp.float32)
        r_ref[0:1, k * _CH:(k + 1) * _CH] = jnp.min(
            c + pen * _BIG, axis=0, keepdims=True)


def _prop_r_to_c(adj_ref, r_ref, c_ref):
    r = r_ref[...]
    changed = jnp.zeros((), jnp.bool_)
    for k in range(_NCH):
        pen = adj_ref[k * _CH:(k + 1) * _CH, :].astype(jnp.float32)
        m = jnp.min(r + pen * _BIG, axis=1, keepdims=True)
        changed |= jnp.any(m != c_ref[k * _CH:(k + 1) * _CH, :])
        c_ref[k * _CH:(k + 1) * _CH, :] = m
    return changed


def _dbscan_kernel(xr_ref, x_ref, d_ref, lab_ref, adj_ref, sqc_ref, c0_ref,
                   r_ref):
    i = pl.program_id(0)

    @pl.when(i == 0)
    def _():
        xx = x_ref[...]
        sqc_ref[...] = lax.dot_general(
            jnp.ones((8, _D), jnp.float32), xx * xx,
            (((1,), (1,)), ((), ())),
            preferred_element_type=jnp.float32,
        )

    xr = xr_ref[...]
    xx = x_ref[...]
    dg = lax.dot_general(xr, xx, (((1,), (1,)), ((), ())),
                         preferred_element_type=jnp.float32)
    sqr = jnp.sum(xr * xr, axis=1, keepdims=True)
    sqc = sqc_ref[0:1, :]
    d2 = jnp.maximum(sqr + sqc - 2.0 * dg, 0.0)
    dblk = jnp.sqrt(d2)
    d_ref[...] = dblk

    pen = (dblk > _EPS).astype(jnp.float32)
    adj_ref[pl.ds(i * _RB, _RB), :] = pen.astype(jnp.int8)

    iota = lax.broadcasted_iota(jnp.int32, (_RB, _N), 1).astype(jnp.float32)
    c0_ref[pl.ds(i * _RB, _RB), :] = jnp.min(
        iota + pen * _BIG, axis=1, keepdims=True)

    @pl.when(i == _G - 1)
    def _():
        def cond(changed):
            return changed

        def body(_):
            _prop_c_to_r(adj_ref, c0_ref, r_ref)
            return _prop_r_to_c(adj_ref, r_ref, c0_ref)

        lax.while_loop(cond, body, jnp.bool_(True))
        lab_ref[...] = c0_ref[...].astype(jnp.int32)


_PT = _N // 16


def _sc_filter_body(lab_hbm, out_hbm, idx2, ones_v, zero_v, cnt_v, out_v,
                    hist_sh):
    cid = lax.axis_index("c")
    tid = lax.axis_index("s")
    base = tid * _PT

    @pl.when(cid == 0)
    def _():
        one16 = jnp.ones((16,), jnp.int32)
        zro16 = jnp.zeros((16,), jnp.int32)
        for g in range(8):
            ones_v[pl.ds(g * 16, 16)] = one16
        for g in range(16):
            zero_v[pl.ds(g * 16, 16)] = zro16

        pltpu.sync_copy(lab_hbm.at[pl.ds(base, 128)], idx2.at[0])
        pltpu.sync_copy(lab_hbm.at[pl.ds(base + 128, 128)], idx2.at[1])
        pltpu.sync_copy(zero_v, hist_sh.at[pl.ds(base, _PT)])

    plsc.subcore_barrier()

    @pl.when(cid == 0)
    def _():
        pltpu.sync_copy(ones_v, hist_sh.at[idx2.at[0]], add=True)
        pltpu.sync_copy(ones_v, hist_sh.at[idx2.at[1]], add=True)

    plsc.subcore_barrier()

    @pl.when(cid == 0)
    def _():
        for j in range(2):
            pltpu.sync_copy(hist_sh.at[idx2.at[j]], cnt_v)
            for g in range(8):
                lab16 = idx2[j, pl.ds(g * 16, 16)]
                sz = cnt_v[pl.ds(g * 16, 16)]
                out_v[pl.ds(j * 128 + g * 16, 16)] = jnp.where(
                    sz >= _MIN_SAMPLES, lab16, -1)
        pltpu.sync_copy(out_v, out_hbm.at[pl.ds(base, _PT)])


def _sc_filter(labels):
    return pl.kernel(
        _sc_filter_body,
        out_type=jax.ShapeDtypeStruct((_N,), jnp.int32),
        mesh=plsc.VectorSubcoreMesh(core_axis_name="c", subcore_axis_name="s"),
        scratch_types=[
            pltpu.VMEM((2, 128), jnp.int32),
            pltpu.VMEM((128,), jnp.int32),
            pltpu.VMEM((_PT,), jnp.int32),
            pltpu.VMEM((128,), jnp.int32),
            pltpu.VMEM((_PT,), jnp.int32),
            pltpu.VMEM_SHARED((_N,), jnp.int32),
        ],
    )(labels)


def kernel(X):
    d, labels = pl.pallas_call(
        _dbscan_kernel,
        grid=(_G,),
        in_specs=[
            pl.BlockSpec((_RB, _D), lambda i: (i, 0)),
            pl.BlockSpec((_N, _D), lambda i: (0, 0)),
        ],
        out_specs=[
            pl.BlockSpec((_RB, _N), lambda i: (i, 0)),
            pl.BlockSpec((_N, 1), lambda i: (0, 0)),
        ],
        out_shape=[
            jax.ShapeDtypeStruct((_N, _N), jnp.float32),
            jax.ShapeDtypeStruct((_N, 1), jnp.int32),
        ],
        scratch_shapes=[
            pltpu.VMEM((_N, _N), jnp.int8),
            pltpu.VMEM((8, _N), jnp.float32),
            pltpu.VMEM((_N, 1), jnp.float32),
            pltpu.VMEM((1, _N), jnp.float32),
        ],
        compiler_params=pltpu.CompilerParams(
            dimension_semantics=("arbitrary",),
        ),
    )(X, X)
    return d, _sc_filter(labels.reshape(_N))


# speedup vs baseline: 2.7394x; 1.1508x over previous
import functools

import jax
import jax.numpy as jnp
from jax import lax
from jax.experimental import pallas as pl
from jax.experimental.pallas import tpu as pltpu
from jax.experimental.pallas import tpu_sc as plsc

_N = 4096
_D = 256
_EPS = 22.0
_MIN_SAMPLES = 5
_RB = 256
_G = _N // _RB
_CH = 512
_NCH = _N // _CH
_BIG = float(_N)


def _prop_c_to_r(adj_ref, c_ref, r_ref):
    c = c_ref[...]
    changed = jnp.zeros((), jnp.bool_)
    for k in range(_NCH):
        pen = adj_ref[:, k * _CH:(k + 1) * _CH].astype(jnp.float32)
        m = jnp.min(c + pen * _BIG, axis=0, keepdims=True)
        changed |= jnp.any(m != r_ref[0:1, k * _CH:(k + 1) * _CH])
        r_ref[0:1, k * _CH:(k + 1) * _CH] = m
    return changed


def _prop_r_to_c(adj_ref, r_ref, c_ref):
    r = r_ref[...]
    changed = jnp.zeros((), jnp.bool_)
    for k in range(_NCH):
        pen = adj_ref[k * _CH:(k + 1) * _CH, :].astype(jnp.float32)
        m = jnp.min(r + pen * _BIG, axis=1, keepdims=True)
        changed |= jnp.any(m != c_ref[k * _CH:(k + 1) * _CH, :])
        c_ref[k * _CH:(k + 1) * _CH, :] = m
    return changed


def _dbscan_kernel(xr_ref, x_ref, d_ref, lab_ref, adj_ref, sqc_ref, c0_ref,
                   r_ref):
    i = pl.program_id(0)

    @pl.when(i == 0)
    def _():
        xx = x_ref[...]
        sqc_ref[...] = lax.dot_general(
            jnp.ones((8, _D), jnp.float32), xx * xx,
            (((1,), (1,)), ((), ())),
            preferred_element_type=jnp.float32,
        )

    xr = xr_ref[...]
    xx = x_ref[...]
    dg = lax.dot_general(xr, xx, (((1,), (1,)), ((), ())),
                         preferred_element_type=jnp.float32)
    sqr = jnp.sum(xr * xr, axis=1, keepdims=True)
    sqc = sqc_ref[0:1, :]
    d2 = jnp.maximum(sqr + sqc - 2.0 * dg, 0.0)
    dblk = jnp.sqrt(d2)
    d_ref[...] = dblk

    pen = (dblk > _EPS).astype(jnp.float32)
    adj_ref[pl.ds(i * _RB, _RB), :] = pen.astype(jnp.int8)

    iota = lax.broadcasted_iota(jnp.int32, (_RB, _N), 1).astype(jnp.float32)
    c0_ref[pl.ds(i * _RB, _RB), :] = jnp.min(
        iota + pen * _BIG, axis=1, keepdims=True)

    @pl.when(i == _G - 1)
    def _():
        r_ref[...] = jnp.full((1, _N), _BIG, jnp.float32)

        def cond(changed):
            return changed

        def body(_):
            ch_r = _prop_c_to_r(adj_ref, c0_ref, r_ref)
            return lax.cond(
                ch_r,
                lambda: _prop_r_to_c(adj_ref, r_ref, c0_ref),
                lambda: jnp.zeros((), jnp.bool_))

        lax.while_loop(cond, body, jnp.bool_(True))
        lab_ref[...] = c0_ref[...].astype(jnp.int32)


_PT = _N // 16


def _sc_filter_body(lab_hbm, out_hbm, idx2, ones_v, zero_v, cnt_v, out_v,
                    hist_sh):
    cid = lax.axis_index("c")
    tid = lax.axis_index("s")
    base = tid * _PT

    @pl.when(cid == 0)
    def _():
        one16 = jnp.ones((16,), jnp.int32)
        zro16 = jnp.zeros((16,), jnp.int32)
        for g in range(8):
            ones_v[pl.ds(g * 16, 16)] = one16
        for g in range(16):
            zero_v[pl.ds(g * 16, 16)] = zro16

        pltpu.sync_copy(lab_hbm.at[pl.ds(base, 128)], idx2.at[0])
        pltpu.sync_copy(lab_hbm.at[pl.ds(base + 128, 128)], idx2.at[1])
        pltpu.sync_copy(zero_v, hist_sh.at[pl.ds(base, _PT)])

    plsc.subcore_barrier()

    @pl.when(cid == 0)
    def _():
        pltpu.sync_copy(ones_v, hist_sh.at[idx2.at[0]], add=True)
        pltpu.sync_copy(ones_v, hist_sh.at[idx2.at[1]], add=True)

    plsc.subcore_barrier()

    @pl.when(cid == 0)
    def _():
        for j in range(2):
            pltpu.sync_copy(hist_sh.at[idx2.at[j]], cnt_v)
            for g in range(8):
                lab16 = idx2[j, pl.ds(g * 16, 16)]
                sz = cnt_v[pl.ds(g * 16, 16)]
                out_v[pl.ds(j * 128 + g * 16, 16)] = jnp.where(
                    sz >= _MIN_SAMPLES, lab16, -1)
        pltpu.sync_copy(out_v, out_hbm.at[pl.ds(base, _PT)])


def _sc_filter(labels):
    return pl.kernel(
        _sc_filter_body,
        out_type=jax.ShapeDtypeStruct((_N,), jnp.int32),
        mesh=plsc.VectorSubcoreMesh(core_axis_name="c", subcore_axis_name="s"),
        scratch_types=[
            pltpu.VMEM((2, 128), jnp.int32),
            pltpu.VMEM((128,), jnp.int32),
            pltpu.VMEM((_PT,), jnp.int32),
            pltpu.VMEM((128,), jnp.int32),
            pltpu.VMEM((_PT,), jnp.int32),
            pltpu.VMEM_SHARED((_N,), jnp.int32),
        ],
    )(labels)


def kernel(X):
    d, labels = pl.pallas_call(
        _dbscan_kernel,
        grid=(_G,),
        in_specs=[
            pl.BlockSpec((_RB, _D), lambda i: (i, 0)),
            pl.BlockSpec((_N, _D), lambda i: (0, 0)),
        ],
        out_specs=[
            pl.BlockSpec((_RB, _N), lambda i: (i, 0)),
            pl.BlockSpec((_N, 1), lambda i: (0, 0)),
        ],
        out_shape=[
            jax.ShapeDtypeStruct((_N, _N), jnp.float32),
            jax.ShapeDtypeStruct((_N, 1), jnp.int32),
        ],
        scratch_shapes=[
            pltpu.VMEM((_N, _N), jnp.int8),
            pltpu.VMEM((8, _N), jnp.float32),
            pltpu.VMEM((_N, 1), jnp.float32),
            pltpu.VMEM((1, _N), jnp.float32),
        ],
        compiler_params=pltpu.CompilerParams(
            dimension_semantics=("arbitrary",),
        ),
    )(X, X)
    return d, _sc_filter(labels.reshape(_N))


# speedup vs baseline: 3.2403x; 1.1829x over previous
import functools

import jax
import jax.numpy as jnp
from jax import lax
from jax.experimental import pallas as pl
from jax.experimental.pallas import tpu as pltpu
from jax.experimental.pallas import tpu_sc as plsc

_N = 4096
_D = 256
_EPS = 22.0
_MIN_SAMPLES = 5
_RB = 256
_G = _N // _RB
_CH = 512
_NCH = _N // _CH
_BIG = float(_N)


_PEN = 8192.0
_RCH = 256


def _prop_c_to_r(adj_ref, c_ref, r_ref):
    acc = jnp.full((_RCH, _N), 2 * _PEN, jnp.float32)
    for k in range(_N // _RCH):
        pen = adj_ref[k * _RCH:(k + 1) * _RCH, :].astype(jnp.float32)
        acc = jnp.minimum(acc, c_ref[k * _RCH:(k + 1) * _RCH, :] + pen)
    m = jnp.min(acc, axis=0, keepdims=True)
    changed = jnp.max(jnp.abs(m - r_ref[...])) > 0.0
    r_ref[...] = m
    return changed


def _prop_r_to_c(adj_ref, r_ref, c_ref):
    acc = jnp.full((_N, _RCH), 2 * _PEN, jnp.float32)
    for k in range(_N // _RCH):
        pen = adj_ref[:, k * _RCH:(k + 1) * _RCH].astype(jnp.float32)
        acc = jnp.minimum(acc, r_ref[0:1, k * _RCH:(k + 1) * _RCH] + pen)
    m = jnp.min(acc, axis=1, keepdims=True)
    changed = jnp.max(jnp.abs(m - c_ref[...])) > 0.0
    c_ref[...] = m
    return changed


def _dbscan_kernel(xr_ref, x_ref, d_ref, lab_ref, adj_ref, sqc_ref, c0_ref,
                   r_ref):
    i = pl.program_id(0)

    @pl.when(i == 0)
    def _():
        xx = x_ref[...]
        sqc_ref[...] = lax.dot_general(
            jnp.ones((8, _D), jnp.float32), xx * xx,
            (((1,), (1,)), ((), ())),
            preferred_element_type=jnp.float32,
        )

    xr = xr_ref[...]
    xx = x_ref[...]
    dg = lax.dot_general(xr, xx, (((1,), (1,)), ((), ())),
                         preferred_element_type=jnp.float32)
    sqr = jnp.sum(xr * xr, axis=1, keepdims=True)
    sqc = sqc_ref[0:1, :]
    d2 = jnp.maximum(sqr + sqc - 2.0 * dg, 0.0)
    dblk = jnp.sqrt(d2)
    d_ref[...] = dblk

    pen = (dblk > _EPS).astype(jnp.float32)
    adj_ref[pl.ds(i * _RB, _RB), :] = (pen * _PEN).astype(jnp.bfloat16)

    iota = lax.broadcasted_iota(jnp.int32, (_RB, _N), 1).astype(jnp.float32)
    c0_ref[pl.ds(i * _RB, _RB), :] = jnp.min(
        iota + pen * _PEN, axis=1, keepdims=True)

    @pl.when(i == _G - 1)
    def _():
        r_ref[...] = jnp.full((1, _N), 2 * _PEN, jnp.float32)

        def cond(changed):
            return changed

        def body(_):
            ch_r = _prop_c_to_r(adj_ref, c0_ref, r_ref)
            return lax.cond(
                ch_r,
                lambda: _prop_r_to_c(adj_ref, r_ref, c0_ref),
                lambda: jnp.zeros((), jnp.bool_))

        lax.while_loop(cond, body, jnp.bool_(True))
        lab_ref[...] = c0_ref[...].astype(jnp.int32)


_PT = _N // 16


def _sc_filter_body(lab_hbm, out_hbm, idx2, ones_v, zero_v, cnt_v, out_v,
                    hist_sh):
    cid = lax.axis_index("c")
    tid = lax.axis_index("s")
    base = tid * _PT

    @pl.when(cid == 0)
    def _():
        one16 = jnp.ones((16,), jnp.int32)
        zro16 = jnp.zeros((16,), jnp.int32)
        for g in range(8):
            ones_v[pl.ds(g * 16, 16)] = one16
        for g in range(16):
            zero_v[pl.ds(g * 16, 16)] = zro16

        pltpu.sync_copy(lab_hbm.at[pl.ds(base, 128)], idx2.at[0])
        pltpu.sync_copy(lab_hbm.at[pl.ds(base + 128, 128)], idx2.at[1])
        pltpu.sync_copy(zero_v, hist_sh.at[pl.ds(base, _PT)])

    plsc.subcore_barrier()

    @pl.when(cid == 0)
    def _():
        pltpu.sync_copy(ones_v, hist_sh.at[idx2.at[0]], add=True)
        pltpu.sync_copy(ones_v, hist_sh.at[idx2.at[1]], add=True)

    plsc.subcore_barrier()

    @pl.when(cid == 0)
    def _():
        for j in range(2):
            pltpu.sync_copy(hist_sh.at[idx2.at[j]], cnt_v)
            for g in range(8):
                lab16 = idx2[j, pl.ds(g * 16, 16)]
                sz = cnt_v[pl.ds(g * 16, 16)]
                out_v[pl.ds(j * 128 + g * 16, 16)] = jnp.where(
                    sz >= _MIN_SAMPLES, lab16, -1)
        pltpu.sync_copy(out_v, out_hbm.at[pl.ds(base, _PT)])


def _sc_filter(labels):
    return pl.kernel(
        _sc_filter_body,
        out_type=jax.ShapeDtypeStruct((_N,), jnp.int32),
        mesh=plsc.VectorSubcoreMesh(core_axis_name="c", subcore_axis_name="s"),
        scratch_types=[
            pltpu.VMEM((2, 128), jnp.int32),
            pltpu.VMEM((128,), jnp.int32),
            pltpu.VMEM((_PT,), jnp.int32),
            pltpu.VMEM((128,), jnp.int32),
            pltpu.VMEM((_PT,), jnp.int32),
            pltpu.VMEM_SHARED((_N,), jnp.int32),
        ],
    )(labels)


def kernel(X):
    d, labels = pl.pallas_call(
        _dbscan_kernel,
        grid=(_G,),
        in_specs=[
            pl.BlockSpec((_RB, _D), lambda i: (i, 0)),
            pl.BlockSpec((_N, _D), lambda i: (0, 0)),
        ],
        out_specs=[
            pl.BlockSpec((_RB, _N), lambda i: (i, 0)),
            pl.BlockSpec((_N, 1), lambda i: (0, 0)),
        ],
        out_shape=[
            jax.ShapeDtypeStruct((_N, _N), jnp.float32),
            jax.ShapeDtypeStruct((_N, 1), jnp.int32),
        ],
        scratch_shapes=[
            pltpu.VMEM((_N, _N), jnp.bfloat16),
            pltpu.VMEM((8, _N), jnp.float32),
            pltpu.VMEM((_N, 1), jnp.float32),
            pltpu.VMEM((1, _N), jnp.float32),
        ],
        compiler_params=pltpu.CompilerParams(
            dimension_semantics=("arbitrary",),
        ),
    )(X, X)
    return d, _sc_filter(labels.reshape(_N))


# speedup vs baseline: 3.3565x; 1.0358x over previous
import functools

import jax
import jax.numpy as jnp
from jax import lax
from jax.experimental import pallas as pl
from jax.experimental.pallas import tpu as pltpu
from jax.experimental.pallas import tpu_sc as plsc

_N = 4096
_D = 256
_EPS = 22.0
_MIN_SAMPLES = 5
_RB = 256
_G = _N // _RB
_CH = 512
_NCH = _N // _CH
_BIG = float(_N)


_PEN = 8192.0
_RCH = 256


def _prop_c_to_r(adj_ref, c_ref, r_ref):
    acc = jnp.full((_RCH, _N), 2 * _PEN, jnp.float32)
    for k in range(_N // _RCH):
        pen = adj_ref[k * _RCH:(k + 1) * _RCH, :].astype(jnp.float32)
        acc = jnp.minimum(acc, c_ref[k * _RCH:(k + 1) * _RCH, :] + pen)
    m = jnp.min(acc, axis=0, keepdims=True)
    changed = jnp.max(jnp.abs(m - r_ref[...])) > 0.0
    r_ref[...] = m
    return changed


def _prop_r_to_c(adj_ref, r_ref, c_ref):
    acc = jnp.full((_N, _RCH), 2 * _PEN, jnp.float32)
    for k in range(_N // _RCH):
        pen = adj_ref[:, k * _RCH:(k + 1) * _RCH].astype(jnp.float32)
        acc = jnp.minimum(acc, r_ref[0:1, k * _RCH:(k + 1) * _RCH] + pen)
    m = jnp.min(acc, axis=1, keepdims=True)
    changed = jnp.max(jnp.abs(m - c_ref[...])) > 0.0
    c_ref[...] = m
    return changed


def _dbscan_kernel(xr_ref, x_ref, d_ref, lab_ref, adj_ref, sqc_ref, c0_ref,
                   r_ref):
    i = pl.program_id(0)

    @pl.when(i == 0)
    def _():
        xx = x_ref[...]
        sqc_ref[...] = lax.dot_general(
            jnp.ones((8, _D), jnp.float32), xx * xx,
            (((1,), (1,)), ((), ())),
            preferred_element_type=jnp.float32,
        )

    xr = xr_ref[...]
    xx = x_ref[...]
    dg = lax.dot_general(xr, xx, (((1,), (1,)), ((), ())),
                         preferred_element_type=jnp.float32)
    sqr = jnp.sum(xr * xr, axis=1, keepdims=True)
    sqc = sqc_ref[0:1, :]
    d2 = jnp.maximum(sqr + sqc - 2.0 * dg, 0.0)
    dblk = jnp.sqrt(d2)
    d_ref[...] = dblk

    pen = jnp.where(d2 > _EPS * _EPS, _PEN, 0.0)
    adj_ref[pl.ds(i * _RB, _RB), :] = pen.astype(jnp.bfloat16)

    iota = lax.broadcasted_iota(jnp.int32, (_RB, _N), 1).astype(jnp.float32)
    c0_ref[pl.ds(i * _RB, _RB), :] = jnp.min(
        iota + pen, axis=1, keepdims=True)

    @pl.when(i == _G - 1)
    def _():
        r_ref[...] = jnp.full((1, _N), 2 * _PEN, jnp.float32)

        def cond(changed):
            return changed

        def body(_):
            ch_r = _prop_c_to_r(adj_ref, c0_ref, r_ref)
            return lax.cond(
                ch_r,
                lambda: _prop_r_to_c(adj_ref, r_ref, c0_ref),
                lambda: jnp.zeros((), jnp.bool_))

        lax.while_loop(cond, body, jnp.bool_(True))
        lab_ref[...] = c0_ref[...].astype(jnp.int32)


_PT = _N // 16


def _sc_filter_body(lab_hbm, out_hbm, idx2, ones_v, zero_v, cnt_v, out_v,
                    hist_sh):
    cid = lax.axis_index("c")
    tid = lax.axis_index("s")
    base = tid * _PT

    @pl.when(cid == 0)
    def _():
        one16 = jnp.ones((16,), jnp.int32)
        zro16 = jnp.zeros((16,), jnp.int32)
        for g in range(8):
            ones_v[pl.ds(g * 16, 16)] = one16
        for g in range(16):
            zero_v[pl.ds(g * 16, 16)] = zro16

        pltpu.sync_copy(lab_hbm.at[pl.ds(base, 128)], idx2.at[0])
        pltpu.sync_copy(lab_hbm.at[pl.ds(base + 128, 128)], idx2.at[1])
        pltpu.sync_copy(zero_v, hist_sh.at[pl.ds(base, _PT)])

    plsc.subcore_barrier()

    @pl.when(cid == 0)
    def _():
        pltpu.sync_copy(ones_v, hist_sh.at[idx2.at[0]], add=True)
        pltpu.sync_copy(ones_v, hist_sh.at[idx2.at[1]], add=True)

    plsc.subcore_barrier()

    @pl.when(cid == 0)
    def _():
        for j in range(2):
            pltpu.sync_copy(hist_sh.at[idx2.at[j]], cnt_v)
            for g in range(8):
                lab16 = idx2[j, pl.ds(g * 16, 16)]
                sz = cnt_v[pl.ds(g * 16, 16)]
                out_v[pl.ds(j * 128 + g * 16, 16)] = jnp.where(
                    sz >= _MIN_SAMPLES, lab16, -1)
        pltpu.sync_copy(out_v, out_hbm.at[pl.ds(base, _PT)])


def _sc_filter(labels):
    return pl.kernel(
        _sc_filter_body,
        out_type=jax.ShapeDtypeStruct((_N,), jnp.int32),
        mesh=plsc.VectorSubcoreMesh(core_axis_name="c", subcore_axis_name="s"),
        scratch_types=[
            pltpu.VMEM((2, 128), jnp.int32),
            pltpu.VMEM((128,), jnp.int32),
            pltpu.VMEM((_PT,), jnp.int32),
            pltpu.VMEM((128,), jnp.int32),
            pltpu.VMEM((_PT,), jnp.int32),
            pltpu.VMEM_SHARED((_N,), jnp.int32),
        ],
    )(labels)


def kernel(X):
    d, labels = pl.pallas_call(
        _dbscan_kernel,
        grid=(_G,),
        in_specs=[
            pl.BlockSpec((_RB, _D), lambda i: (i, 0)),
            pl.BlockSpec((_N, _D), lambda i: (0, 0)),
        ],
        out_specs=[
            pl.BlockSpec((_RB, _N), lambda i: (i, 0)),
            pl.BlockSpec((_N, 1), lambda i: (0, 0)),
        ],
        out_shape=[
            jax.ShapeDtypeStruct((_N, _N), jnp.float32),
            jax.ShapeDtypeStruct((_N, 1), jnp.int32),
        ],
        scratch_shapes=[
            pltpu.VMEM((_N, _N), jnp.bfloat16),
            pltpu.VMEM((8, _N), jnp.float32),
            pltpu.VMEM((_N, 1), jnp.float32),
            pltpu.VMEM((1, _N), jnp.float32),
        ],
        compiler_params=pltpu.CompilerParams(
            dimension_semantics=("arbitrary",),
        ),
    )(X, X)
    return d, _sc_filter(labels.reshape(_N))


# speedup vs baseline: 3.4770x; 1.0359x over previous
import functools

import jax
import jax.numpy as jnp
from jax import lax
from jax.experimental import pallas as pl
from jax.experimental.pallas import tpu as pltpu
from jax.experimental.pallas import tpu_sc as plsc

_N = 4096
_D = 256
_EPS = 22.0
_MIN_SAMPLES = 5
_RB = 256
_G = _N // _RB
_CH = 512
_NCH = _N // _CH
_BIG = float(_N)


_PEN = 8192.0
_RCH = 256


def _prop_c_to_r(adj_ref, c_ref, r_ref):
    acc = jnp.full((_RCH, _N), 2 * _PEN, jnp.float32)
    for k in range(_N // _RCH):
        pen = adj_ref[k * _RCH:(k + 1) * _RCH, :].astype(jnp.float32)
        acc = jnp.minimum(acc, c_ref[k * _RCH:(k + 1) * _RCH, :] + pen)
    m = jnp.min(acc, axis=0, keepdims=True)
    changed = jnp.max(jnp.abs(m - r_ref[...])) > 0.0
    r_ref[...] = m
    return changed


def _prop_r_to_c(adj_ref, r_ref, c_ref):
    acc = jnp.full((_N, _RCH), 2 * _PEN, jnp.float32)
    for k in range(_N // _RCH):
        pen = adj_ref[:, k * _RCH:(k + 1) * _RCH].astype(jnp.float32)
        acc = jnp.minimum(acc, r_ref[0:1, k * _RCH:(k + 1) * _RCH] + pen)
    m = jnp.min(acc, axis=1, keepdims=True)
    changed = jnp.max(jnp.abs(m - c_ref[...])) > 0.0
    c_ref[...] = m
    return changed


def _dbscan_kernel(x_ref, d_ref, lab_ref, adj_ref, sqc_ref, c0_ref,
                   r_ref):
    i = pl.program_id(0)

    @pl.when(i == 0)
    def _():
        xx = x_ref[...]
        sqc_ref[...] = lax.dot_general(
            jnp.ones((8, _D), jnp.float32), xx * xx,
            (((1,), (1,)), ((), ())),
            preferred_element_type=jnp.float32,
        )

    xx = x_ref[...]
    xr = x_ref[pl.ds(i * _RB, _RB), :]
    dg = lax.dot_general(xr, xx, (((1,), (1,)), ((), ())),
                         preferred_element_type=jnp.float32)
    sqr = jnp.sum(xr * xr, axis=1, keepdims=True)
    sqc = sqc_ref[0:1, :]
    d2 = jnp.maximum(sqr + sqc - 2.0 * dg, 0.0)
    dblk = jnp.sqrt(d2)
    d_ref[...] = dblk

    pen = jnp.where(d2 > _EPS * _EPS, _PEN, 0.0)
    adj_ref[pl.ds(i * _RB, _RB), :] = pen.astype(jnp.bfloat16)

    iota = lax.broadcasted_iota(jnp.int32, (_RB, _N), 1).astype(jnp.float32)
    c0_ref[pl.ds(i * _RB, _RB), :] = jnp.min(
        iota + pen, axis=1, keepdims=True)

    @pl.when(i == _G - 1)
    def _():
        r_ref[...] = jnp.full((1, _N), 2 * _PEN, jnp.float32)

        def cond(changed):
            return changed

        def body(_):
            ch_r = _prop_c_to_r(adj_ref, c0_ref, r_ref)
            return lax.cond(
                ch_r,
                lambda: _prop_r_to_c(adj_ref, r_ref, c0_ref),
                lambda: jnp.zeros((), jnp.bool_))

        lax.while_loop(cond, body, jnp.bool_(True))
        lab_ref[...] = r_ref[...].astype(jnp.int32)


_PT = _N // 16


def _sc_filter_body(lab_hbm, out_hbm, idx2, ones_v, zero_v, cnt_v, out_v,
                    hist_sh):
    cid = lax.axis_index("c")
    tid = lax.axis_index("s")
    base = tid * _PT

    @pl.when(cid == 0)
    def _():
        one16 = jnp.ones((16,), jnp.int32)
        zro16 = jnp.zeros((16,), jnp.int32)
        for g in range(8):
            ones_v[pl.ds(g * 16, 16)] = one16
        for g in range(16):
            zero_v[pl.ds(g * 16, 16)] = zro16

        pltpu.sync_copy(lab_hbm.at[pl.ds(base, 128)], idx2.at[0])
        pltpu.sync_copy(lab_hbm.at[pl.ds(base + 128, 128)], idx2.at[1])
        pltpu.sync_copy(zero_v, hist_sh.at[pl.ds(base, _PT)])

    plsc.subcore_barrier()

    @pl.when(cid == 0)
    def _():
        pltpu.sync_copy(ones_v, hist_sh.at[idx2.at[0]], add=True)
        pltpu.sync_copy(ones_v, hist_sh.at[idx2.at[1]], add=True)

    plsc.subcore_barrier()

    @pl.when(cid == 0)
    def _():
        for j in range(2):
            pltpu.sync_copy(hist_sh.at[idx2.at[j]], cnt_v)
            for g in range(8):
                lab16 = idx2[j, pl.ds(g * 16, 16)]
                sz = cnt_v[pl.ds(g * 16, 16)]
                out_v[pl.ds(j * 128 + g * 16, 16)] = jnp.where(
                    sz >= _MIN_SAMPLES, lab16, -1)
        pltpu.sync_copy(out_v, out_hbm.at[pl.ds(base, _PT)])


def _sc_filter(labels):
    return pl.kernel(
        _sc_filter_body,
        out_type=jax.ShapeDtypeStruct((_N,), jnp.int32),
        mesh=plsc.VectorSubcoreMesh(core_axis_name="c", subcore_axis_name="s"),
        scratch_types=[
            pltpu.VMEM((2, 128), jnp.int32),
            pltpu.VMEM((128,), jnp.int32),
            pltpu.VMEM((_PT,), jnp.int32),
            pltpu.VMEM((128,), jnp.int32),
            pltpu.VMEM((_PT,), jnp.int32),
            pltpu.VMEM_SHARED((_N,), jnp.int32),
        ],
    )(labels)


def kernel(X):
    d, labels = pl.pallas_call(
        _dbscan_kernel,
        grid=(_G,),
        in_specs=[
            pl.BlockSpec((_N, _D), lambda i: (0, 0)),
        ],
        out_specs=[
            pl.BlockSpec((_RB, _N), lambda i: (i, 0)),
            pl.BlockSpec((1, _N), lambda i: (0, 0)),
        ],
        out_shape=[
            jax.ShapeDtypeStruct((_N, _N), jnp.float32),
            jax.ShapeDtypeStruct((1, _N), jnp.int32),
        ],
        scratch_shapes=[
            pltpu.VMEM((_N, _N), jnp.bfloat16),
            pltpu.VMEM((8, _N), jnp.float32),
            pltpu.VMEM((_N, 1), jnp.float32),
            pltpu.VMEM((1, _N), jnp.float32),
        ],
        compiler_params=pltpu.CompilerParams(
            dimension_semantics=("arbitrary",),
        ),
    )(X)
    return d, _sc_filter(labels.reshape(_N))
